# baseline (speedup 1.0000x reference)
import functools

import jax
import jax.numpy as jnp
from jax import lax
from jax.experimental import pallas as pl
from jax.experimental.pallas import tpu as pltpu
from jax.experimental.pallas import tpu_sc as plsc

D = 128
SCALE = float(D) ** 0.5
NW = 32
CHUNK = 128
LANES = 16
NBUF = 7
PREF = 5


@functools.partial(jax.jit, static_argnums=(2,))
def _gather_scale(emb_var, idx_flat, n_chunks):
  B = NW * n_chunks * CHUNK
  per_w = n_chunks * CHUNK
  mesh = plsc.VectorSubcoreMesh(core_axis_name="c", subcore_axis_name="s")

  @functools.partial(
      pl.kernel,
      mesh=mesh,
      out_type=jax.ShapeDtypeStruct((B, D), jnp.float32),
      scratch_types=[
          pltpu.VMEM((per_w,), jnp.int32),
          [pltpu.VMEM((CHUNK, D), jnp.float32) for _ in range(NBUF)],
          [pltpu.SemaphoreType.DMA for _ in range(NBUF)],
          [pltpu.SemaphoreType.DMA for _ in range(NBUF)],
      ],
  )
  def k(table_hbm, idx_hbm, out_hbm, idx_v, bufs, gsems, ssems):
    wid = lax.axis_index("s") * 2 + lax.axis_index("c")
    base = wid * per_w
    pltpu.sync_copy(idx_hbm.at[pl.ds(base, per_w)], idx_v)

    def gather(j, buf, gsem):
      off = pl.multiple_of(j * CHUNK, 8)
      pltpu.async_copy(table_hbm.at[idx_v.at[pl.ds(off, CHUNK)]], buf, gsem)

    def scale_buf(buf):
      def srows(ri, carry):
        r0 = ri * 8
        for dr in range(8):
          for c in range(D // LANES):
            sl = pl.ds(c * LANES, LANES)
            buf[r0 + dr, sl] = buf[r0 + dr, sl] * SCALE
        return carry

      lax.fori_loop(0, CHUNK // 8, srows, 0)

    def chunk_body(j, b, guard):
      if guard:
        @pl.when(j + PREF < n_chunks)
        def _():
          @pl.when(j >= NBUF - PREF)
          def _():
            pltpu.make_async_copy(
                bufs[(b + PREF) % NBUF],
                out_hbm.at[pl.ds(0, CHUNK)],
                ssems[(b + PREF) % NBUF],
            ).wait()

          gather(j + PREF, bufs[(b + PREF) % NBUF], gsems[(b + PREF) % NBUF])
      pltpu.make_async_copy(
          table_hbm.at[pl.ds(0, CHUNK)], bufs[b], gsems[b]
      ).wait()
      scale_buf(bufs[b])
      pltpu.async_copy(
          bufs[b], out_hbm.at[pl.ds(base + j * CHUNK, CHUNK)], ssems[b]
      )

    for t in range(PREF):
      gather(t, bufs[t], gsems[t])

    n_main = (n_chunks // NBUF) * NBUF

    def outer(jo, carry):
      for b in range(NBUF):
        chunk_body(jo * NBUF + b, b, True)
      return carry

    lax.fori_loop(0, n_chunks // NBUF, outer, 0)
    for t in range(n_main, n_chunks):
      chunk_body(t, t % NBUF, t + PREF < n_chunks)

    for t in range(n_chunks - NBUF, n_chunks):
      pltpu.make_async_copy(
          bufs[t % NBUF], out_hbm.at[pl.ds(0, CHUNK)], ssems[t % NBUF]
      ).wait()

  return k(emb_var, idx_flat)


def kernel(ids, emb_var):
  batch, seq = ids.shape
  idx_flat = ids.T.astype(jnp.int32).reshape(-1)
  n_chunks = batch * seq // (NW * CHUNK)
  out = _gather_scale(emb_var, idx_flat, n_chunks)
  return out.reshape(seq, batch, D).transpose(1, 0, 2)
